# baseline (speedup 1.0000x reference)
import functools

import jax
import jax.numpy as jnp
from jax import lax
from jax.experimental import pallas as pl
from jax.experimental.pallas import tpu as pltpu
from jax.experimental.pallas import tpu_sc as plsc

_USER = 50000
_ITEM = 50000
_N = _USER + _ITEM
_E = 3200000
_D = 16
_LAYERS = 3

_NC = 2
_NS = 16
_NW = _NC * _NS
_NP = 100352
_CHUNK = 256
_SUP = 2
_BLK = _SUP * _CHUNK
_EW = 100352
_NB = _EW // _BLK
_EW_LAST = _E - 31 * _EW
_NB_LAST = _EW_LAST // _BLK
_RT = _NP // _NS


def _spmm_body(adj_hbm, val_hbm, table_hbm, out_hbm,
               src_b0, src_b1, val_b0, val_b1,
               d00, d01, d10, d11,
               rows_b0, rows_b1, acc_sh, sem_l, sem_g, sem_s):
    c = lax.axis_index("c")
    s = lax.axis_index("s")
    wid = c * _NS + s

    src_b = (src_b0, src_b1)
    val_b = (val_b0, val_b1)
    dst_b = ((d00, d01), (d10, d11))
    rows_b = (rows_b0, rows_b1)

    ebase = wid * _EW
    nb = jnp.where(wid == _NW - 1, _NB_LAST, _NB)

    def fire_linear(bi, p):
        e0 = ebase + bi * _BLK
        pltpu.make_async_copy(adj_hbm.at[1, pl.ds(e0, _BLK)], src_b[p],
                              sem_l).start()
        pltpu.make_async_copy(val_hbm.at[pl.ds(e0, _BLK)], val_b[p],
                              sem_l).start()
        for j in range(_SUP):
            pltpu.make_async_copy(adj_hbm.at[0, pl.ds(e0 + j * _CHUNK,
                                                      _CHUNK)],
                                  dst_b[p][j], sem_l).start()

    def drain_linear(bi, p):
        e0 = ebase + bi * _BLK
        pltpu.make_async_copy(adj_hbm.at[1, pl.ds(e0, _BLK)], src_b[p],
                              sem_l).wait()
        pltpu.make_async_copy(val_hbm.at[pl.ds(e0, _BLK)], val_b[p],
                              sem_l).wait()
        for j in range(_SUP):
            pltpu.make_async_copy(adj_hbm.at[0, pl.ds(e0 + j * _CHUNK,
                                                      _CHUNK)],
                                  dst_b[p][j], sem_l).wait()

    def fire_gathers(p):
        for j in range(_SUP):
            pltpu.make_async_copy(
                table_hbm.at[src_b[p].at[pl.ds(j * _CHUNK, _CHUNK)]],
                rows_b[p].at[pl.ds(j * _CHUNK, _CHUNK)], sem_g).start()

    def drain_gathers(p):
        for j in range(_SUP):
            pltpu.make_async_copy(
                table_hbm.at[src_b[p].at[pl.ds(j * _CHUNK, _CHUNK)]],
                rows_b[p].at[pl.ds(j * _CHUNK, _CHUNK)], sem_g).wait()

    def fire_scatters(p):
        for j in range(_SUP):
            pltpu.make_async_copy(rows_b[p].at[pl.ds(j * _CHUNK, _CHUNK)],
                                  acc_sh.at[dst_b[p][j]],
                                  sem_s).start(add=True)

    def drain_scatters(p):
        for j in range(_SUP):
            pltpu.make_async_copy(rows_b[p].at[pl.ds(j * _CHUNK, _CHUNK)],
                                  acc_sh.at[dst_b[p][j]],
                                  sem_s).wait()

    def scale(p):
        for g in range(_BLK // 16):
            vv16 = val_b[p][pl.ds(g * 16, 16)]
            for e in range(16):
                bb = lax.gather(
                    vv16, jnp.full((16, 1), e, jnp.int32),
                    lax.GatherDimensionNumbers(
                        offset_dims=(), collapsed_slice_dims=(0,),
                        start_index_map=(0,)),
                    slice_sizes=(1,),
                    mode=lax.GatherScatterMode.PROMISE_IN_BOUNDS)
                idx = g * 16 + e
                rows_b[p][idx] = rows_b[p][idx] * bb

    fire_linear(0, 0)
    drain_linear(0, 0)
    fire_gathers(0)
    zero = jnp.zeros((_D,), jnp.float32)
    def zset(r, carry):
        rows_b1[r] = zero
        return carry
    lax.fori_loop(0, _BLK, zset, 0)
    for z in range(_RT // _BLK):
        pltpu.sync_copy(rows_b1, acc_sh.at[pl.ds(s * _RT + z * _BLK, _BLK)])
    _ztail = _RT - (_RT // _BLK) * _BLK
    if _ztail:
        pltpu.sync_copy(rows_b1.at[pl.ds(0, _ztail)],
                        acc_sh.at[pl.ds(s * _RT + (_RT // _BLK) * _BLK,
                                        _ztail)])
    plsc.subcore_barrier()

    def body(k, carry):
        for p in (0, 1):
            bi = 2 * k + p

            @pl.when(bi > 0)
            def _():
                drain_scatters(1 - p)

            @pl.when(bi < nb - 1)
            def _():
                fire_linear(bi + 1, 1 - p)

            drain_gathers(p)
            scale(p)

            @pl.when(bi < nb - 1)
            def _():
                drain_linear(bi + 1, 1 - p)
                fire_gathers(1 - p)

            fire_scatters(p)
        return carry

    lax.fori_loop(0, nb // 2, body, 0)
    drain_scatters(1)

    plsc.subcore_barrier()
    pltpu.sync_copy(acc_sh.at[pl.ds(s * _RT, _RT)],
                    out_hbm.at[pl.ds(c * _NP + s * _RT, _RT)])


_spmm = functools.partial(
    pl.kernel,
    out_type=jax.ShapeDtypeStruct((_NC * _NP, _D), jnp.float32),
    mesh=plsc.VectorSubcoreMesh(core_axis_name="c", subcore_axis_name="s"),
    compiler_params=pltpu.CompilerParams(use_tc_tiling_on_sc=False),
    scratch_types=[
        pltpu.VMEM((_BLK,), jnp.int32),
        pltpu.VMEM((_BLK,), jnp.int32),
        pltpu.VMEM((_BLK,), jnp.float32),
        pltpu.VMEM((_BLK,), jnp.float32),
        pltpu.VMEM((_CHUNK,), jnp.int32),
        pltpu.VMEM((_CHUNK,), jnp.int32),
        pltpu.VMEM((_CHUNK,), jnp.int32),
        pltpu.VMEM((_CHUNK,), jnp.int32),
        pltpu.VMEM((_BLK, _D), jnp.float32),
        pltpu.VMEM((_BLK, _D), jnp.float32),
        pltpu.VMEM_SHARED((_NP, _D), jnp.float32),
        pltpu.SemaphoreType.DMA,
        pltpu.SemaphoreType.DMA,
        pltpu.SemaphoreType.DMA,
    ],
)(_spmm_body)


_R32 = _NP // _NW
_MC = 8
_MR = _R32 // _MC


def _scmerge_body(parts_hbm, tot_hbm, table_hbm, tot2_hbm,
                  p0b, p1b, ttb, sem):
    c = lax.axis_index("c")
    s = lax.axis_index("s")
    base = (c * _NS + s) * _R32

    def step(i, carry):
        r0 = base + i * _MR
        pltpu.make_async_copy(parts_hbm.at[pl.ds(r0, _MR)], p0b, sem).start()
        pltpu.make_async_copy(parts_hbm.at[pl.ds(_NP + r0, _MR)], p1b,
                              sem).start()
        pltpu.make_async_copy(tot_hbm.at[pl.ds(r0, _MR)], ttb, sem).start()
        pltpu.make_async_copy(parts_hbm.at[pl.ds(r0, _MR)], p0b, sem).wait()
        pltpu.make_async_copy(parts_hbm.at[pl.ds(_NP + r0, _MR)], p1b,
                              sem).wait()
        pltpu.make_async_copy(tot_hbm.at[pl.ds(r0, _MR)], ttb, sem).wait()
        for r in range(_MR):
            t = p0b[r] + p1b[r]
            p0b[r] = t
            ttb[r] = ttb[r] + t
        pltpu.sync_copy(p0b, table_hbm.at[pl.ds(r0, _MR)])
        pltpu.sync_copy(ttb, tot2_hbm.at[pl.ds(r0, _MR)])
        return carry

    lax.fori_loop(0, _MC, step, 0)


_scmerge = functools.partial(
    pl.kernel,
    out_type=[jax.ShapeDtypeStruct((_NP, _D), jnp.float32),
              jax.ShapeDtypeStruct((_NP, _D), jnp.float32)],
    mesh=plsc.VectorSubcoreMesh(core_axis_name="c", subcore_axis_name="s"),
    compiler_params=pltpu.CompilerParams(use_tc_tiling_on_sc=False),
    scratch_types=[
        pltpu.VMEM((_MR, _D), jnp.float32),
        pltpu.VMEM((_MR, _D), jnp.float32),
        pltpu.VMEM((_MR, _D), jnp.float32),
        pltpu.SemaphoreType.DMA,
    ],
)(_scmerge_body)


def _prep_body(u_hbm, i_hbm, inip_hbm, buf, sem):
    c = lax.axis_index("c")
    s = lax.axis_index("s")
    w = c * _NS + s
    base = w * _R32

    def copy_rows(src_ref, s0, d0, n):
        pltpu.sync_copy(src_ref.at[pl.ds(s0, n)], buf.at[pl.ds(0, n)])
        pltpu.sync_copy(buf.at[pl.ds(0, n)], inip_hbm.at[pl.ds(d0, n)])

    @pl.when(w < 15)
    def _():
        for t in range(4):
            copy_rows(u_hbm, base + t * 784, base + t * 784, 784)

    @pl.when(w == 15)
    def _():
        for t in range(3):
            copy_rows(u_hbm, base + t * 784, base + t * 784, 784)
        copy_rows(u_hbm, base + 2352, base + 2352, 608)
        copy_rows(i_hbm, 0, _USER, 176)

    @pl.when(jnp.logical_and(w > 15, w < 31))
    def _():
        for t in range(4):
            copy_rows(i_hbm, base - _USER + t * 784, base + t * 784, 784)

    @pl.when(w == 31)
    def _():
        for t in range(3):
            copy_rows(i_hbm, base - _USER + t * 784, base + t * 784, 784)
        copy_rows(i_hbm, base - _USER + 2352, base + 2352, 432)
        zero = jnp.zeros((_D,), jnp.float32)

        def zset(r, carry):
            buf[r] = zero
            return carry

        lax.fori_loop(0, _NP - _N, zset, 0)
        pltpu.sync_copy(buf.at[pl.ds(0, _NP - _N)],
                        inip_hbm.at[pl.ds(_N, _NP - _N)])


_prep = functools.partial(
    pl.kernel,
    out_type=jax.ShapeDtypeStruct((_NP, _D), jnp.float32),
    mesh=plsc.VectorSubcoreMesh(core_axis_name="c", subcore_axis_name="s"),
    compiler_params=pltpu.CompilerParams(use_tc_tiling_on_sc=False),
    scratch_types=[
        pltpu.VMEM((784, _D), jnp.float32),
        pltpu.SemaphoreType.DMA,
    ],
)(_prep_body)


def kernel(adj_indices, adj_values, uEmbeds, iEmbeds):
    inip = _prep(uEmbeds, iEmbeds)
    table = inip
    tot = inip
    for _ in range(_LAYERS):
        parts = _spmm(adj_indices, adj_values, table)
        table, tot = _scmerge(parts, tot)
    return tot[:_USER], tot[_USER:_N]


# speedup vs baseline: 1.0054x; 1.0054x over previous
import functools

import jax
import jax.numpy as jnp
from jax import lax
from jax.experimental import pallas as pl
from jax.experimental.pallas import tpu as pltpu
from jax.experimental.pallas import tpu_sc as plsc

_USER = 50000
_ITEM = 50000
_N = _USER + _ITEM
_E = 3200000
_D = 16
_LAYERS = 3

_NC = 2
_NS = 16
_NW = _NC * _NS
_NP = 100352
_CHUNK = 256
_SUP = 2
_BLK = _SUP * _CHUNK
_EW = 100352
_NB = _EW // _BLK
_EW_LAST = _E - 31 * _EW
_NB_LAST = _EW_LAST // _BLK
_RT = _NP // _NS


def _spmm_body(src_hbm, dst_hbm, val_hbm, table_hbm, out_hbm,
               src_b0, src_b1, val_b0, val_b1,
               d00, d01, d10, d11,
               rows_b0, rows_b1, acc_sh, sem_l, sem_g, sem_s):
    c = lax.axis_index("c")
    s = lax.axis_index("s")
    wid = c * _NS + s

    src_b = (src_b0, src_b1)
    val_b = (val_b0, val_b1)
    dst_b = ((d00, d01), (d10, d11))
    rows_b = (rows_b0, rows_b1)

    ebase = wid * _EW
    nb = jnp.where(wid == _NW - 1, _NB_LAST, _NB)

    def fire_linear(bi, p):
        e0 = ebase + bi * _BLK
        pltpu.make_async_copy(src_hbm.at[pl.ds(e0, _BLK)], src_b[p],
                              sem_l).start()
        pltpu.make_async_copy(val_hbm.at[pl.ds(e0, _BLK)], val_b[p],
                              sem_l).start()
        for j in range(_SUP):
            pltpu.make_async_copy(dst_hbm.at[pl.ds(e0 + j * _CHUNK, _CHUNK)],
                                  dst_b[p][j], sem_l).start()

    def drain_linear(bi, p):
        e0 = ebase + bi * _BLK
        pltpu.make_async_copy(src_hbm.at[pl.ds(e0, _BLK)], src_b[p],
                              sem_l).wait()
        pltpu.make_async_copy(val_hbm.at[pl.ds(e0, _BLK)], val_b[p],
                              sem_l).wait()
        for j in range(_SUP):
            pltpu.make_async_copy(dst_hbm.at[pl.ds(e0 + j * _CHUNK, _CHUNK)],
                                  dst_b[p][j], sem_l).wait()

    def fire_gathers(p):
        for j in range(_SUP):
            pltpu.make_async_copy(
                table_hbm.at[src_b[p].at[pl.ds(j * _CHUNK, _CHUNK)]],
                rows_b[p].at[pl.ds(j * _CHUNK, _CHUNK)], sem_g).start()

    def drain_gathers(p):
        for j in range(_SUP):
            pltpu.make_async_copy(
                table_hbm.at[src_b[p].at[pl.ds(j * _CHUNK, _CHUNK)]],
                rows_b[p].at[pl.ds(j * _CHUNK, _CHUNK)], sem_g).wait()

    def fire_scatters(p):
        for j in range(_SUP):
            pltpu.make_async_copy(rows_b[p].at[pl.ds(j * _CHUNK, _CHUNK)],
                                  acc_sh.at[dst_b[p][j]],
                                  sem_s).start(add=True)

    def drain_scatters(p):
        for j in range(_SUP):
            pltpu.make_async_copy(rows_b[p].at[pl.ds(j * _CHUNK, _CHUNK)],
                                  acc_sh.at[dst_b[p][j]],
                                  sem_s).wait()

    def scale(p):
        for g in range(_BLK // 16):
            vv16 = val_b[p][pl.ds(g * 16, 16)]
            for e in range(16):
                bb = lax.gather(
                    vv16, jnp.full((16, 1), e, jnp.int32),
                    lax.GatherDimensionNumbers(
                        offset_dims=(), collapsed_slice_dims=(0,),
                        start_index_map=(0,)),
                    slice_sizes=(1,),
                    mode=lax.GatherScatterMode.PROMISE_IN_BOUNDS)
                idx = g * 16 + e
                rows_b[p][idx] = rows_b[p][idx] * bb

    fire_linear(0, 0)
    drain_linear(0, 0)
    fire_gathers(0)
    zero = jnp.zeros((_D,), jnp.float32)
    def zset(r, carry):
        rows_b1[r] = zero
        return carry
    lax.fori_loop(0, _BLK, zset, 0)
    for z in range(_RT // _BLK):
        pltpu.sync_copy(rows_b1, acc_sh.at[pl.ds(s * _RT + z * _BLK, _BLK)])
    _ztail = _RT - (_RT // _BLK) * _BLK
    if _ztail:
        pltpu.sync_copy(rows_b1.at[pl.ds(0, _ztail)],
                        acc_sh.at[pl.ds(s * _RT + (_RT // _BLK) * _BLK,
                                        _ztail)])
    plsc.subcore_barrier()

    def body(k, carry):
        for p in (0, 1):
            bi = 2 * k + p

            @pl.when(bi > 0)
            def _():
                drain_scatters(1 - p)

            @pl.when(bi < nb - 1)
            def _():
                fire_linear(bi + 1, 1 - p)

            drain_gathers(p)
            scale(p)

            @pl.when(bi < nb - 1)
            def _():
                drain_linear(bi + 1, 1 - p)
                fire_gathers(1 - p)

            fire_scatters(p)
        return carry

    lax.fori_loop(0, nb // 2, body, 0)
    drain_scatters(1)

    plsc.subcore_barrier()
    pltpu.sync_copy(acc_sh.at[pl.ds(s * _RT, _RT)],
                    out_hbm.at[pl.ds(c * _NP + s * _RT, _RT)])


_spmm = functools.partial(
    pl.kernel,
    out_type=jax.ShapeDtypeStruct((_NC * _NP, _D), jnp.float32),
    mesh=plsc.VectorSubcoreMesh(core_axis_name="c", subcore_axis_name="s"),
    compiler_params=pltpu.CompilerParams(use_tc_tiling_on_sc=False),
    scratch_types=[
        pltpu.VMEM((_BLK,), jnp.int32),
        pltpu.VMEM((_BLK,), jnp.int32),
        pltpu.VMEM((_BLK,), jnp.float32),
        pltpu.VMEM((_BLK,), jnp.float32),
        pltpu.VMEM((_CHUNK,), jnp.int32),
        pltpu.VMEM((_CHUNK,), jnp.int32),
        pltpu.VMEM((_CHUNK,), jnp.int32),
        pltpu.VMEM((_CHUNK,), jnp.int32),
        pltpu.VMEM((_BLK, _D), jnp.float32),
        pltpu.VMEM((_BLK, _D), jnp.float32),
        pltpu.VMEM_SHARED((_NP, _D), jnp.float32),
        pltpu.SemaphoreType.DMA,
        pltpu.SemaphoreType.DMA,
        pltpu.SemaphoreType.DMA,
    ],
)(_spmm_body)


_R32 = _NP // _NW
_MC = 8
_MR = _R32 // _MC


def _scmerge_body(parts_hbm, tot_hbm, table_hbm, tot2_hbm,
                  p0b, p1b, ttb, sem):
    c = lax.axis_index("c")
    s = lax.axis_index("s")
    base = (c * _NS + s) * _R32

    def step(i, carry):
        r0 = base + i * _MR
        pltpu.make_async_copy(parts_hbm.at[pl.ds(r0, _MR)], p0b, sem).start()
        pltpu.make_async_copy(parts_hbm.at[pl.ds(_NP + r0, _MR)], p1b,
                              sem).start()
        pltpu.make_async_copy(tot_hbm.at[pl.ds(r0, _MR)], ttb, sem).start()
        pltpu.make_async_copy(parts_hbm.at[pl.ds(r0, _MR)], p0b, sem).wait()
        pltpu.make_async_copy(parts_hbm.at[pl.ds(_NP + r0, _MR)], p1b,
                              sem).wait()
        pltpu.make_async_copy(tot_hbm.at[pl.ds(r0, _MR)], ttb, sem).wait()
        for r in range(_MR):
            t = p0b[r] + p1b[r]
            p0b[r] = t
            ttb[r] = ttb[r] + t
        pltpu.sync_copy(p0b, table_hbm.at[pl.ds(r0, _MR)])
        pltpu.sync_copy(ttb, tot2_hbm.at[pl.ds(r0, _MR)])
        return carry

    lax.fori_loop(0, _MC, step, 0)


_scmerge = functools.partial(
    pl.kernel,
    out_type=[jax.ShapeDtypeStruct((_NP, _D), jnp.float32),
              jax.ShapeDtypeStruct((_NP, _D), jnp.float32)],
    mesh=plsc.VectorSubcoreMesh(core_axis_name="c", subcore_axis_name="s"),
    compiler_params=pltpu.CompilerParams(use_tc_tiling_on_sc=False),
    scratch_types=[
        pltpu.VMEM((_MR, _D), jnp.float32),
        pltpu.VMEM((_MR, _D), jnp.float32),
        pltpu.VMEM((_MR, _D), jnp.float32),
        pltpu.SemaphoreType.DMA,
    ],
)(_scmerge_body)


def kernel(adj_indices, adj_values, uEmbeds, iEmbeds):
    dst = adj_indices[0]
    src = adj_indices[1]

    ini = jnp.concatenate([uEmbeds, iEmbeds], axis=0)
    inip = jnp.concatenate([ini, jnp.zeros((_NP - _N, _D), jnp.float32)])
    table = inip
    tot = inip
    for _ in range(_LAYERS):
        parts = _spmm(src, dst, adj_values, table)
        table, tot = _scmerge(parts, tot)
    return tot[:_USER], tot[_USER:_N]
